# baseline (speedup 1.0000x reference)
import jax
import jax.numpy as jnp
from jax.experimental import pallas as pl
from jax.experimental.pallas import tpu as pltpu


def _cdiv(a: int, b: int) -> int:
    return (a + b - 1) // b


def _mlp_kernel(x_ref, w0_ref, b0_ref, w1_ref, b1_ref, w2_ref, b2_ref, o_ref):
    rows = x_ref.shape[0] // 2
    for s in range(2):
        sl = pl.ds(s * rows, rows)
        h = x_ref[sl, :]
        h = jnp.dot(h, w0_ref[...], preferred_element_type=jnp.float32)
        h = jnp.maximum(h + b0_ref[...], 0.0)
        h = jnp.dot(h, w1_ref[...], preferred_element_type=jnp.float32)
        h = jnp.maximum(h + b1_ref[...], 0.0)
        h = jnp.dot(h, w2_ref[...], preferred_element_type=jnp.float32)
        o_ref[sl, :] = h + b2_ref[...]


def kernel(x, w0, b0, w1, b1, w2, b2, *, batch_tile: int = 4096):
    B, Din = x.shape
    D1 = w0.shape[1]
    D2 = w1.shape[1]
    Dout = w2.shape[1]

    TB = min(batch_tile, B)
    grid = _cdiv(B, TB)

    b0r = b0.reshape(1, D1)
    b1r = b1.reshape(1, D2)
    b2r = b2.reshape(1, Dout)

    resident = lambda i: (0, 0)
    return pl.pallas_call(
        _mlp_kernel,
        out_shape=jax.ShapeDtypeStruct((B, Dout), x.dtype),
        grid=(grid,),
        in_specs=[
            pl.BlockSpec((TB, Din), lambda i: (i, 0)),
            pl.BlockSpec((Din, D1), resident),
            pl.BlockSpec((1, D1), resident),
            pl.BlockSpec((D1, D2), resident),
            pl.BlockSpec((1, D2), resident),
            pl.BlockSpec((D2, Dout), resident),
            pl.BlockSpec((1, Dout), resident),
        ],
        out_specs=pl.BlockSpec((TB, Dout), lambda i: (i, 0)),
        compiler_params=pltpu.CompilerParams(
            dimension_semantics=("parallel",),
            vmem_limit_bytes=100 * 1024 * 1024,
        ),
    )(x, w0, b0r, w1, b1r, w2, b2r)


# speedup vs baseline: 1.0291x; 1.0291x over previous
import jax
import jax.numpy as jnp
from jax.experimental import pallas as pl
from jax.experimental.pallas import tpu as pltpu


def _cdiv(a: int, b: int) -> int:
    return (a + b - 1) // b


def _mlp_kernel(x_ref, w0_ref, b0_ref, w1_ref, b1_ref, w2_ref, b2_ref, o_ref):
    h = x_ref[...]
    h = jnp.dot(h, w0_ref[...], preferred_element_type=jnp.float32)
    h = jnp.maximum(h + b0_ref[...], 0.0)
    h = jnp.dot(h, w1_ref[...], preferred_element_type=jnp.float32)
    h = jnp.maximum(h + b1_ref[...], 0.0)
    h = jnp.dot(h, w2_ref[...], preferred_element_type=jnp.float32)
    o_ref[...] = h + b2_ref[...]


def kernel(x, w0, b0, w1, b1, w2, b2, *, batch_tile: int = 2048):
    B, Din = x.shape
    D1 = w0.shape[1]
    D2 = w1.shape[1]
    Dout = w2.shape[1]

    TB = min(batch_tile, B)
    grid = _cdiv(B, TB)

    b0r = b0.reshape(1, D1)
    b1r = b1.reshape(1, D2)
    b2r = b2.reshape(1, Dout)

    resident = lambda i: (0, 0)
    return pl.pallas_call(
        _mlp_kernel,
        out_shape=jax.ShapeDtypeStruct((B, Dout), x.dtype),
        grid=(grid,),
        in_specs=[
            pl.BlockSpec((TB, Din), lambda i: (i, 0)),
            pl.BlockSpec((Din, D1), resident),
            pl.BlockSpec((1, D1), resident),
            pl.BlockSpec((D1, D2), resident),
            pl.BlockSpec((1, D2), resident),
            pl.BlockSpec((D2, Dout), resident),
            pl.BlockSpec((1, Dout), resident),
        ],
        out_specs=pl.BlockSpec((TB, Dout), lambda i: (i, 0)),
        compiler_params=pltpu.CompilerParams(
            dimension_semantics=("parallel",),
            vmem_limit_bytes=100 * 1024 * 1024,
        ),
    )(x, w0, b0r, w1, b1r, w2, b2r)
